# baseline (speedup 1.0000x reference)
import functools

import jax
import jax.numpy as jnp
from jax import lax
from jax.experimental import pallas as pl
from jax.experimental.pallas import tpu as pltpu
from jax.experimental.pallas import tpu_sc as plsc

D = 128
B_TOTAL = 4096 * 200
NC, NS = 2, 16
NW = NC * NS
BPW = B_TOTAL // NW
SUB = 128
CMAX = 384
NFULL = BPW // CMAX
CTAIL = BPW - NFULL * CMAX
NCH = NFULL + 1
ROWS_X = B_TOTAL // SUB

_mesh = plsc.VectorSubcoreMesh(core_axis_name="c", subcore_axis_name="s")


def _nsub(g):
    return (CTAIL // SUB) if g == NCH - 1 else (CMAX // SUB)


@functools.partial(
    pl.kernel,
    mesh=_mesh,
    out_type=jax.ShapeDtypeStruct((B_TOTAL, D), jnp.float32),
    scratch_types=[
        pltpu.VMEM((BPW // SUB, SUB), jnp.int32),
        pltpu.VMEM((CMAX, D), jnp.float32),
        pltpu.VMEM((CMAX, D), jnp.float32),
        pltpu.SemaphoreType.DMA,
        pltpu.SemaphoreType.DMA,
        pltpu.SemaphoreType.DMA,
        pltpu.SemaphoreType.DMA,
    ],
)
def _emb_lookup(x_hbm, tab_hbm, out_hbm,
                idx_all, rows0, rows1, gsem0, gsem1, osem0, osem1):
    wid = lax.axis_index("s") * NC + lax.axis_index("c")
    rx = wid * (BPW // SUB)

    rows = (rows0, rows1)
    gsem = (gsem0, gsem1)
    osem = (osem0, osem1)
    RPC = CMAX // SUB

    pltpu.sync_copy(x_hbm.at[pl.ds(rx, BPW // SUB)], idx_all)

    def fire_gather(g, b, nsub=RPC):
        for j in range(nsub):
            pltpu.async_copy(
                tab_hbm.at[idx_all.at[g * RPC + j]],
                rows[b].at[pl.ds(j * SUB, SUB)], gsem[b]
            )

    def drain_gather(g, b, nsub=RPC):
        for j in range(nsub):
            pltpu.make_async_copy(
                tab_hbm.at[idx_all.at[g * RPC + j]],
                rows[b].at[pl.ds(j * SUB, SUB)], gsem[b]
            ).wait()

    def fire_store(g, b, c=CMAX):
        pltpu.async_copy(
            rows[b].at[pl.ds(0, c)], out_hbm.at[pl.ds((rx + g * RPC) * SUB, c)],
            osem[b]
        )

    def drain_store(g, b, c=CMAX):
        pltpu.make_async_copy(
            rows[b].at[pl.ds(0, c)], out_hbm.at[pl.ds((rx + g * RPC) * SUB, c)],
            osem[b]
        ).wait()

    def ring_iter(g, b, first=False):
        drain_gather(g - 1, 1 - b)
        fire_store(g - 1, 1 - b)
        if not first:
            drain_store(g - 2, b)
        fire_gather(g, b)

    fire_gather(0, 0)
    ring_iter(1, 1, first=True)
    ring_iter(2, 0)

    def body(p, carry):
        g = 2 * p + 3
        ring_iter(g, 1)
        ring_iter(g + 1, 0)
        return carry

    lax.fori_loop(0, (NFULL - 4) // 2, body, 0)

    ring_iter(NCH - 2, 1)
    drain_gather(NCH - 2, 1)
    fire_store(NCH - 2, 1)
    drain_store(NCH - 3, 0)
    fire_gather(NCH - 1, 0, nsub=CTAIL // SUB)
    drain_gather(NCH - 1, 0, nsub=CTAIL // SUB)
    fire_store(NCH - 1, 0, c=CTAIL)
    drain_store(NCH - 2, 1)
    drain_store(NCH - 1, 0, c=CTAIL)


def kernel(x, table):
    xf = x.astype(jnp.int32).reshape(ROWS_X, SUB)
    out = _emb_lookup(xf, table)
    return out.reshape(x.shape[0], x.shape[1], D)
